# Initial kernel scaffold; baseline (speedup 1.0000x reference)
import jax
import jax.numpy as jnp
from jax.experimental import pallas as pl


def kernel(x, w_q, w_k, w_v, b_q, b_k, b_v, w_proj, w_exp, rep_mat, shift_stack, w_out, out_bias):
    raise NotImplementedError("write your pallas kernel here")


# speedup vs baseline: 6.0718x; 6.0718x over previous
import functools

import jax
import jax.numpy as jnp
from jax import lax
from jax.experimental import pallas as pl
from jax.experimental.pallas import tpu as pltpu

_N = 32
_C = 16
_KD = 8
_D = 16
_IMG = 16
_H = 2
_BT = 8
_GROUPS = 2
_BSTEP = _BT * _GROUPS


def _body(x_ref, wbig_ref, bbig_ref, wp_ref, rep2_ref, wexp_ref, shift2_ref,
          bd_ref, biasT_ref, o_ref):
    f32 = jnp.float32
    x = x_ref[...]
    qkv = jnp.dot(x, wbig_ref[...], preferred_element_type=f32) + bbig_ref[...]

    rows = _BT * _N
    rowb = lax.broadcasted_iota(jnp.int32, (rows, rows), 0) // _N
    colb = lax.broadcasted_iota(jnp.int32, (rows, rows), 1) // _N
    mask = jnp.where(rowb == colb, 0.0, -1e30).astype(f32)

    outs = []
    for g in range(_GROUPS):
        r0 = g * rows
        os = []
        for h in range(_H):
            q = qkv[r0:r0 + rows, 8 * h:8 * h + 8]
            k = qkv[r0:r0 + rows, 16 + 8 * h:24 + 8 * h]
            v = qkv[r0:r0 + rows, 32 + 16 * h:48 + 16 * h]
            s = lax.dot_general(q, k, (((1,), (1,)), ((), ())),
                                preferred_element_type=f32)
            s = s + mask
            s = s - jnp.max(s, axis=-1, keepdims=True)
            p = jnp.exp(s)
            p = p * pl.reciprocal(jnp.sum(p, axis=-1, keepdims=True),
                                  approx=True)
            os.append(jnp.dot(p, v, preferred_element_type=f32))
        ocat = jnp.concatenate(os, axis=1)
        acc_att = jnp.dot(ocat, wp_ref[...],
                          preferred_element_type=f32)

        vcat = qkv[r0:r0 + rows, 32:64]
        v0 = vcat.reshape(_BT, _N, 32)[:, :_IMG, :].reshape(_BT * _IMG, 32)
        v0 = v0 * jnp.clip(v0 + 3.0, 0.0, 6.0) * (1.0 / 6.0)
        lhs = jnp.dot(v0, rep2_ref[...],
                      preferred_element_type=f32) * wexp_ref[...]
        conv = jnp.dot(lhs, shift2_ref[...],
                       preferred_element_type=f32)

        cat = jnp.concatenate([acc_att, conv[:, :_IMG], conv[:, _IMG:]],
                              axis=0)
        outs.append(jnp.dot(bd_ref[...], cat,
                            preferred_element_type=f32) + biasT_ref[...])
    outT = jnp.concatenate(outs, axis=0)
    o_ref[...] = outT.reshape(_BSTEP, _C, _IMG)


@jax.jit
def kernel(x, w_q, w_k, w_v, b_q, b_k, b_v, w_proj, w_exp, rep_mat,
           shift_stack, w_out, out_bias):
    B, N, C = x.shape
    f32 = jnp.float32

    wbig = jnp.concatenate([w_q[0], w_q[1], w_k[0], w_k[1], w_v[0], w_v[1]],
                           axis=1)
    bbig = jnp.concatenate([b_q[0, 0], b_q[1, 0], b_k[0, 0], b_k[1, 0],
                            b_v[0, 0], b_v[1, 0]])[None, :]
    wp = jnp.concatenate([w_proj[0], w_proj[1]], axis=0)

    eye2 = jnp.eye(2, dtype=f32)
    rep2 = jnp.kron(eye2, rep_mat)
    shift2 = jnp.kron(eye2, shift_stack)
    wexp = jnp.tile(jnp.concatenate([w_exp[0], w_exp[1]], axis=1),
                    (_BT, 1))

    woutT = w_out.T
    eyeb = jnp.eye(_BT, dtype=f32)
    bd = jnp.concatenate([jnp.kron(eyeb, woutT),
                          jnp.kron(eyeb, woutT[:, :_IMG]),
                          jnp.kron(eyeb, woutT[:, _IMG:])],
                         axis=1)
    biasT = jnp.tile(out_bias.T, (_BT, 1))

    x2 = x.reshape(B * N, C)
    steps = B // _BSTEP
    const = lambda g: (0, 0)
    outT = pl.pallas_call(
        _body,
        out_shape=jax.ShapeDtypeStruct((B, _C, _IMG), f32),
        grid=(steps,),
        in_specs=[
            pl.BlockSpec((_BSTEP * _N, C), lambda g: (g, 0)),
            pl.BlockSpec(wbig.shape, const),
            pl.BlockSpec(bbig.shape, const),
            pl.BlockSpec(wp.shape, const),
            pl.BlockSpec(rep2.shape, const),
            pl.BlockSpec(wexp.shape, const),
            pl.BlockSpec(shift2.shape, const),
            pl.BlockSpec(bd.shape, const),
            pl.BlockSpec(biasT.shape, const),
        ],
        out_specs=pl.BlockSpec((_BSTEP, _C, _IMG), lambda g: (g, 0, 0)),
        compiler_params=pltpu.CompilerParams(
            dimension_semantics=("parallel",)),
    )(x2, wbig, bbig, wp, rep2, wexp, shift2, bd, biasT)
    return jnp.swapaxes(outT, 1, 2)


# speedup vs baseline: 11.7636x; 1.9374x over previous
import functools

import jax
import jax.numpy as jnp
from jax import lax
from jax.experimental import pallas as pl
from jax.experimental.pallas import tpu as pltpu

_N = 32
_C = 16
_KD = 8
_IMG = 16
_H = 2
_BT = 8
_GROUPS = 4
_BSTEP = _BT * _GROUPS


def _body(x_ref, wbig_ref, bbig_ref, wp_ref, rep2_ref, wexp_ref, shift2_ref,
          bd_ref, biasT_ref, mask_ref, o_ref):
    f32 = jnp.float32
    bf16 = jnp.bfloat16
    x = x_ref[...]
    qkv = jnp.dot(x, wbig_ref[...], preferred_element_type=f32) + bbig_ref[...]
    qkv = qkv.astype(bf16)

    rows = _BT * _N
    mask = mask_ref[...]

    os = []
    for g in range(_GROUPS):
        r0 = g * rows
        for h in range(_H):
            q = qkv[r0:r0 + rows, 32 + 8 * h:40 + 8 * h]
            k = qkv[r0:r0 + rows, 48 + 8 * h:56 + 8 * h]
            v = qkv[r0:r0 + rows, 16 * h:16 * h + 16]
            s = lax.dot_general(q, k, (((1,), (1,)), ((), ())),
                                preferred_element_type=f32)
            p = jnp.exp(s + mask)
            r = jnp.sum(p, axis=-1, keepdims=True)
            o = jnp.dot(p.astype(bf16), v, preferred_element_type=f32)
            os.append(o * pl.reciprocal(r, approx=True))
    ocat = jnp.concatenate(
        [jnp.concatenate(os[2 * g:2 * g + 2], axis=1) for g in range(_GROUPS)],
        axis=0).astype(bf16)
    acc_att = jnp.dot(ocat, wp_ref[...],
                      preferred_element_type=f32)

    v0 = qkv[:, 0:32].reshape(_BSTEP, _N, 32)[:, :_IMG, :]
    v0 = v0.reshape(_BSTEP * _IMG, 32)
    v0 = v0 * jnp.clip(v0 + 3.0, 0.0, 6.0).astype(bf16)
    lhs = jnp.dot(v0, rep2_ref[...], preferred_element_type=f32)
    lhs = lhs.astype(bf16) * wexp_ref[...]
    conv = jnp.dot(lhs, shift2_ref[...],
                   preferred_element_type=f32)

    outs = []
    for g in range(_GROUPS):
        cg = conv[g * _BT * _IMG:(g + 1) * _BT * _IMG]
        cat = jnp.concatenate([acc_att[g * rows:(g + 1) * rows],
                               cg[:, :_IMG], cg[:, _IMG:]],
                              axis=0).astype(bf16)
        outs.append(jnp.dot(bd_ref[...], cat,
                            preferred_element_type=f32))
    outT = jnp.concatenate(outs, axis=0) + biasT_ref[...]
    o_ref[...] = outT.reshape(_BSTEP, _C, _IMG)


@jax.jit
def kernel(x, w_q, w_k, w_v, b_q, b_k, b_v, w_proj, w_exp, rep_mat,
           shift_stack, w_out, out_bias):
    B, N, C = x.shape
    f32 = jnp.float32
    bf16 = jnp.bfloat16

    wbig = jnp.concatenate([w_v[0], w_v[1], w_q[0], w_q[1], w_k[0], w_k[1]],
                           axis=1).astype(bf16)
    bbig = jnp.concatenate([b_v[0, 0], b_v[1, 0], b_q[0, 0], b_q[1, 0],
                            b_k[0, 0], b_k[1, 0]])[None, :]
    wp = jnp.concatenate([w_proj[0], w_proj[1]], axis=0).astype(bf16)

    eye2 = jnp.eye(2, dtype=f32)
    rep2 = jnp.kron(eye2, rep_mat).astype(bf16)
    shift2 = jnp.kron(eye2, shift_stack).astype(bf16)
    wexp = jnp.tile(jnp.concatenate([w_exp[0], w_exp[1]], axis=1) * (1.0 / 6.0),
                    (_BSTEP, 1)).astype(bf16)

    woutT = w_out.T
    eyeb = jnp.eye(_BT, dtype=f32)
    bd = jnp.concatenate([jnp.kron(eyeb, woutT),
                          jnp.kron(eyeb, woutT[:, :_IMG]),
                          jnp.kron(eyeb, woutT[:, _IMG:])],
                         axis=1).astype(bf16)
    biasT = jnp.tile(out_bias.T, (_BSTEP, 1))

    rows = _BT * _N
    bi = jnp.arange(rows, dtype=jnp.int32) // _N
    mask = jnp.where(bi[:, None] == bi[None, :], 0.0, -1e30).astype(f32)

    x2 = x.reshape(B * N, C).astype(bf16)
    steps = B // _BSTEP
    const = lambda g: (0, 0)
    outT = pl.pallas_call(
        _body,
        out_shape=jax.ShapeDtypeStruct((B, _C, _IMG), f32),
        grid=(steps,),
        in_specs=[
            pl.BlockSpec((_BSTEP * _N, C), lambda g: (g, 0)),
            pl.BlockSpec(wbig.shape, const),
            pl.BlockSpec(bbig.shape, const),
            pl.BlockSpec(wp.shape, const),
            pl.BlockSpec(rep2.shape, const),
            pl.BlockSpec(wexp.shape, const),
            pl.BlockSpec(shift2.shape, const),
            pl.BlockSpec(bd.shape, const),
            pl.BlockSpec(biasT.shape, const),
            pl.BlockSpec(mask.shape, const),
        ],
        out_specs=pl.BlockSpec((_BSTEP, _C, _IMG), lambda g: (g, 0, 0)),
        compiler_params=pltpu.CompilerParams(
            dimension_semantics=("parallel",)),
    )(x2, wbig, bbig, wp, rep2, wexp, shift2, bd, biasT, mask)
    return jnp.swapaxes(outT, 1, 2)
